# baseline (speedup 1.0000x reference)
import functools

import jax
import jax.numpy as jnp
from jax import lax
from jax.experimental import pallas as pl
from jax.experimental.pallas import tpu as pltpu
from jax.experimental.pallas import tpu_sc as plsc

N_USER = 30000
N_ITEM = 20000
N_NODES = N_USER + N_ITEM
NNZ = 800000
DIM = 64
NBRANCH = 3
DIMC = DIM * NBRANCH
CCH = 32
NCH = DIMC // CCH
N_LAYERS = 4
EPS = 1e-8

NC = 2
NS = 16
LANES = 16

K_EDGE = 512
SUB = K_EDGE // 128
E_PER_TILE = 51200
G_BIG = E_PER_TILE // K_EDGE
E_PAD = E_PER_TILE * NS
ROWS_MAIN = 3128
ROWS_LAST = N_NODES - (NS - 1) * ROWS_MAIN


def _item_branch(feat, W, b):
    M, Kd = feat.shape
    BM = 400

    def body(f_ref, w_ref, b_ref, o1_ref, o2_ref):
        t = jnp.tanh(
            jnp.dot(f_ref[...], w_ref[...], preferred_element_type=jnp.float32)
            + b_ref[...]
        )
        o1_ref[...] = t[:, :CCH]
        o2_ref[...] = t[:, CCH:]

    ospec = pl.BlockSpec((BM, CCH), lambda i: (i, 0))
    return pl.pallas_call(
        body,
        grid=(M // BM,),
        in_specs=[
            pl.BlockSpec((BM, Kd), lambda i: (i, 0)),
            pl.BlockSpec((Kd, DIM), lambda i: (0, 0)),
            pl.BlockSpec((1, DIM), lambda i: (0, 0)),
        ],
        out_specs=[ospec, ospec],
        out_shape=[jax.ShapeDtypeStruct((M, CCH), jnp.float32)] * 2,
    )(feat, W, b.reshape(1, DIM))


def _user_part(user_embeddings, id_prompt, v_prompt, t_prompt):
    BM = 400

    def body(u_ref, pid_ref, pv_ref, pt_ref, *o_refs):
        u = u_ref[...]
        for b, p_ref in enumerate((pid_ref, pv_ref, pt_ref)):
            ps = jnp.sum(p_ref[...], axis=0, keepdims=True)
            o_refs[2 * b][...] = u[:, :CCH] + ps[:, :CCH]
            o_refs[2 * b + 1][...] = u[:, CCH:] + ps[:, CCH:]

    pspec = pl.BlockSpec((3, DIM), lambda i: (0, 0))
    ospec = pl.BlockSpec((BM, CCH), lambda i: (i, 0))
    return pl.pallas_call(
        body,
        grid=(N_USER // BM,),
        in_specs=[pl.BlockSpec((BM, DIM), lambda i: (i, 0)), pspec, pspec, pspec],
        out_specs=[ospec] * NCH,
        out_shape=[jax.ShapeDtypeStruct((N_USER, CCH), jnp.float32)] * NCH,
    )(user_embeddings, id_prompt, v_prompt, t_prompt)


def _cosine_acc(y_chunks, ego_chunks, acc):
    BM = 400

    def body(*refs):
        y_refs = refs[0:NCH]
        e_refs = refs[NCH:2 * NCH]
        a_ref = refs[2 * NCH]
        yw_refs = refs[2 * NCH + 1:2 * NCH + 1 + NCH]
        ao_ref = refs[2 * NCH + 1 + NCH]
        parts = []
        for b in range(NBRANCH):
            ys = jnp.concatenate([y_refs[2 * b][...], y_refs[2 * b + 1][...]], axis=1)
            es = jnp.concatenate([e_refs[2 * b][...], e_refs[2 * b + 1][...]], axis=1)
            dot = jnp.sum(ys * es, axis=1, keepdims=True)
            ny = jnp.sqrt(jnp.sum(ys * ys, axis=1, keepdims=True))
            ne = jnp.sqrt(jnp.sum(es * es, axis=1, keepdims=True))
            w = dot / (jnp.maximum(ny, EPS) * jnp.maximum(ne, EPS))
            yw = w * ys
            yw_refs[2 * b][...] = yw[:, :CCH]
            yw_refs[2 * b + 1][...] = yw[:, CCH:]
            parts.append(yw)
        ao_ref[...] = a_ref[...] + jnp.concatenate(parts, axis=1)

    cspec = pl.BlockSpec((BM, CCH), lambda i: (i, 0))
    aspec = pl.BlockSpec((BM, DIMC), lambda i: (i, 0))
    outs = pl.pallas_call(
        body,
        grid=(N_NODES // BM,),
        in_specs=[cspec] * NCH + [cspec] * NCH + [aspec],
        out_specs=[cspec] * NCH + [aspec],
        out_shape=[jax.ShapeDtypeStruct((N_NODES, CCH), jnp.float32)] * NCH
        + [jax.ShapeDtypeStruct((N_NODES, DIMC), jnp.float32)],
    )(*y_chunks, *ego_chunks, acc)
    return list(outs[:NCH]), outs[NCH]


def _spmm_pair(xa, xb, cols2d, rows2d, vals1d, zeros_nc):
    mesh = plsc.VectorSubcoreMesh(
        core_axis_name="c", subcore_axis_name="s", num_cores=NC, num_subcores=NS
    )

    @functools.partial(
        pl.kernel,
        out_type=[jax.ShapeDtypeStruct((N_NODES, CCH), jnp.float32)] * 2,
        mesh=mesh,
        compiler_params=pltpu.CompilerParams(use_tc_tiling_on_sc=False),
        scratch_types=[
            pltpu.VMEM_SHARED((N_NODES, CCH), jnp.float32),
            pltpu.VMEM((SUB, 128), jnp.int32),
            pltpu.VMEM((SUB, 128), jnp.int32),
            pltpu.VMEM((K_EDGE,), jnp.float32),
            pltpu.VMEM((K_EDGE, CCH), jnp.float32),
            pltpu.SemaphoreType.DMA((SUB,)),
        ],
    )
    def spmm_kernel(xa_h, xb_h, cols_h, rows_h, vals_h, z_h, outa_h, outb_h,
                    acc, cols_v, rows_v, vals_v, stage, sem):
        s = lax.axis_index("s")

        def run(x_h, out_h):
            rsl_main = pl.ds(s * ROWS_MAIN, ROWS_MAIN)
            rsl_last = pl.ds((NS - 1) * ROWS_MAIN, ROWS_LAST)

            @pl.when(s < NS - 1)
            def _():
                pltpu.sync_copy(z_h.at[rsl_main], acc.at[rsl_main])

            @pl.when(s == NS - 1)
            def _():
                pltpu.sync_copy(z_h.at[rsl_last], acc.at[rsl_last])

            plsc.subcore_barrier()

            def gbody(g, carry):
                r0 = s * (E_PER_TILE // 128) + g * SUB
                pltpu.sync_copy(cols_h.at[pl.ds(r0, SUB)], cols_v)
                pltpu.sync_copy(rows_h.at[pl.ds(r0, SUB)], rows_v)
                pltpu.sync_copy(
                    vals_h.at[pl.ds(s * E_PER_TILE + g * K_EDGE, K_EDGE)], vals_v
                )
                handles = [
                    pltpu.async_copy(
                        x_h.at[cols_v.at[j]], stage.at[pl.ds(j * 128, 128)], sem.at[j]
                    )
                    for j in range(SUB)
                ]

                def sbody(gg, c2):
                    val16 = vals_v[pl.ds(gg * LANES, LANES)]
                    for i in range(LANES):
                        e = gg * LANES + i
                        v = lax.gather(
                            val16,
                            jnp.full((LANES, 1), i, jnp.int32),
                            dimension_numbers=lax.GatherDimensionNumbers(
                                offset_dims=(),
                                collapsed_slice_dims=(0,),
                                start_index_map=(0,),
                            ),
                            slice_sizes=(1,),
                            mode=lax.GatherScatterMode.PROMISE_IN_BOUNDS,
                        )
                        for h in range(CCH // LANES):
                            csl = pl.ds(h * LANES, LANES)
                            stage[e, csl] = stage[e, csl] * v
                    return c2

                for j in range(SUB):
                    handles[j].wait()
                    gpj = 128 // LANES
                    lax.fori_loop(j * gpj, (j + 1) * gpj, sbody, 0, unroll=4)
                    pltpu.sync_copy(
                        stage.at[pl.ds(j * 128, 128)], acc.at[rows_v.at[j]], add=True
                    )
                return carry

            lax.fori_loop(0, G_BIG, gbody, 0)
            plsc.subcore_barrier()

            @pl.when(s < NS - 1)
            def _():
                pltpu.sync_copy(acc.at[rsl_main], out_h.at[rsl_main])

            @pl.when(s == NS - 1)
            def _():
                pltpu.sync_copy(acc.at[rsl_last], out_h.at[rsl_last])

        c = lax.axis_index("c")

        @pl.when(c == 0)
        def _():
            run(xa_h, outa_h)

        @pl.when(c == 1)
        def _():
            run(xb_h, outb_h)

    return spmm_kernel(xa, xb, cols2d, rows2d, vals1d, zeros_nc)


def kernel(adj_indices, adj_values, user_embeddings, item_embeddings, v_feat, t_feat, id_prompt, v_prompt, t_prompt, W_id, b_id, W_v, b_v, W_t, b_t):
    pad = E_PAD - NNZ
    rows = jnp.concatenate([adj_indices[0], jnp.zeros((pad,), jnp.int32)])
    cols = jnp.concatenate([adj_indices[1], jnp.zeros((pad,), jnp.int32)])
    vals = jnp.concatenate([adj_values, jnp.zeros((pad,), jnp.float32)])
    rows2d = rows.reshape(E_PAD // 128, 128)
    cols2d = cols.reshape(E_PAD // 128, 128)
    zeros_nc = jnp.zeros((N_NODES, CCH), jnp.float32)

    u_chunks = _user_part(user_embeddings, id_prompt, v_prompt, t_prompt)
    i_id = _item_branch(item_embeddings, W_id, b_id)
    i_v = _item_branch(v_feat, W_v, b_v)
    i_t = _item_branch(t_feat, W_t, b_t)
    i_chunks = [i_id[0], i_id[1], i_v[0], i_v[1], i_t[0], i_t[1]]
    ego_chunks = [
        jnp.concatenate([u, it], axis=0) for u, it in zip(u_chunks, i_chunks)
    ]
    ego = jnp.concatenate(ego_chunks, axis=1)

    acc = ego
    y_chunks = ego_chunks
    for _ in range(N_LAYERS):
        s_chunks = [None] * NCH
        for k in range(NCH // NC):
            sa, sb = _spmm_pair(
                y_chunks[k], y_chunks[k + NCH // NC],
                cols2d, rows2d, vals, zeros_nc,
            )
            s_chunks[k] = sa
            s_chunks[k + NCH // NC] = sb
        y_chunks, acc = _cosine_acc(s_chunks, ego_chunks, acc)
    return acc[:N_USER], acc[N_USER:]


# speedup vs baseline: 1.1440x; 1.1440x over previous
import functools

import jax
import jax.numpy as jnp
from jax import lax
from jax.experimental import pallas as pl
from jax.experimental.pallas import tpu as pltpu
from jax.experimental.pallas import tpu_sc as plsc

N_USER = 30000
N_ITEM = 20000
N_NODES = N_USER + N_ITEM
NNZ = 800000
DIM = 64
NBRANCH = 3
DIMC = DIM * NBRANCH
CCH = 32
NCH = DIMC // CCH
N_LAYERS = 4
EPS = 1e-8

NC = 2
NS = 16
LANES = 16

K_EDGE = 512
SUB = K_EDGE // 128
QCH = 4
E_PER_TILE = 51200
G_BIG = E_PER_TILE // K_EDGE
E_PAD = E_PER_TILE * NS
ROWS_MAIN = 3128
ROWS_LAST = N_NODES - (NS - 1) * ROWS_MAIN


def _item_branch(feat, W, b):
    M, Kd = feat.shape
    BM = 400

    def body(f_ref, w_ref, b_ref, o1_ref, o2_ref):
        t = jnp.tanh(
            jnp.dot(f_ref[...], w_ref[...], preferred_element_type=jnp.float32)
            + b_ref[...]
        )
        o1_ref[...] = t[:, :CCH]
        o2_ref[...] = t[:, CCH:]

    ospec = pl.BlockSpec((BM, CCH), lambda i: (i, 0))
    return pl.pallas_call(
        body,
        grid=(M // BM,),
        in_specs=[
            pl.BlockSpec((BM, Kd), lambda i: (i, 0)),
            pl.BlockSpec((Kd, DIM), lambda i: (0, 0)),
            pl.BlockSpec((1, DIM), lambda i: (0, 0)),
        ],
        out_specs=[ospec, ospec],
        out_shape=[jax.ShapeDtypeStruct((M, CCH), jnp.float32)] * 2,
    )(feat, W, b.reshape(1, DIM))


def _user_part(user_embeddings, id_prompt, v_prompt, t_prompt):
    BM = 400

    def body(u_ref, pid_ref, pv_ref, pt_ref, *o_refs):
        u = u_ref[...]
        for b, p_ref in enumerate((pid_ref, pv_ref, pt_ref)):
            ps = jnp.sum(p_ref[...], axis=0, keepdims=True)
            o_refs[2 * b][...] = u[:, :CCH] + ps[:, :CCH]
            o_refs[2 * b + 1][...] = u[:, CCH:] + ps[:, CCH:]

    pspec = pl.BlockSpec((3, DIM), lambda i: (0, 0))
    ospec = pl.BlockSpec((BM, CCH), lambda i: (i, 0))
    return pl.pallas_call(
        body,
        grid=(N_USER // BM,),
        in_specs=[pl.BlockSpec((BM, DIM), lambda i: (i, 0)), pspec, pspec, pspec],
        out_specs=[ospec] * NCH,
        out_shape=[jax.ShapeDtypeStruct((N_USER, CCH), jnp.float32)] * NCH,
    )(user_embeddings, id_prompt, v_prompt, t_prompt)


def _cosine_acc(y_chunks, ego_chunks, acc):
    BM = 400

    def body(*refs):
        y_refs = refs[0:NCH]
        e_refs = refs[NCH:2 * NCH]
        a_ref = refs[2 * NCH]
        yw_refs = refs[2 * NCH + 1:2 * NCH + 1 + NCH]
        ao_ref = refs[2 * NCH + 1 + NCH]
        parts = []
        for b in range(NBRANCH):
            ys = jnp.concatenate([y_refs[2 * b][...], y_refs[2 * b + 1][...]], axis=1)
            es = jnp.concatenate([e_refs[2 * b][...], e_refs[2 * b + 1][...]], axis=1)
            dot = jnp.sum(ys * es, axis=1, keepdims=True)
            ny = jnp.sqrt(jnp.sum(ys * ys, axis=1, keepdims=True))
            ne = jnp.sqrt(jnp.sum(es * es, axis=1, keepdims=True))
            w = dot / (jnp.maximum(ny, EPS) * jnp.maximum(ne, EPS))
            yw = w * ys
            yw_refs[2 * b][...] = yw[:, :CCH]
            yw_refs[2 * b + 1][...] = yw[:, CCH:]
            parts.append(yw)
        ao_ref[...] = a_ref[...] + jnp.concatenate(parts, axis=1)

    cspec = pl.BlockSpec((BM, CCH), lambda i: (i, 0))
    aspec = pl.BlockSpec((BM, DIMC), lambda i: (i, 0))
    outs = pl.pallas_call(
        body,
        grid=(N_NODES // BM,),
        in_specs=[cspec] * NCH + [cspec] * NCH + [aspec],
        out_specs=[cspec] * NCH + [aspec],
        out_shape=[jax.ShapeDtypeStruct((N_NODES, CCH), jnp.float32)] * NCH
        + [jax.ShapeDtypeStruct((N_NODES, DIMC), jnp.float32)],
    )(*y_chunks, *ego_chunks, acc)
    return list(outs[:NCH]), outs[NCH]


def _spmm_pair(xa, xb, cols2d, rows2d, vals1d, zeros_nc):
    mesh = plsc.VectorSubcoreMesh(
        core_axis_name="c", subcore_axis_name="s", num_cores=NC, num_subcores=NS
    )

    @functools.partial(
        pl.kernel,
        out_type=[jax.ShapeDtypeStruct((N_NODES, CCH), jnp.float32)] * 2,
        mesh=mesh,
        compiler_params=pltpu.CompilerParams(use_tc_tiling_on_sc=False),
        scratch_types=[
            pltpu.VMEM_SHARED((N_NODES, CCH), jnp.float32),
            pltpu.VMEM((QCH, K_EDGE), jnp.int32),
            pltpu.VMEM((QCH * SUB, 128), jnp.int32),
            pltpu.VMEM((QCH, K_EDGE), jnp.float32),
            pltpu.VMEM((K_EDGE, CCH), jnp.float32),
            pltpu.SemaphoreType.DMA((2,)),
        ],
    )
    def spmm_kernel(xa_h, xb_h, cols_h, rows_h, vals_h, z_h, outa_h, outb_h,
                    acc, cols_v, rows_v, vals_v, stage, sem):
        s = lax.axis_index("s")

        def run(x_h, out_h):
            rsl_main = pl.ds(s * ROWS_MAIN, ROWS_MAIN)
            rsl_last = pl.ds((NS - 1) * ROWS_MAIN, ROWS_LAST)

            @pl.when(s < NS - 1)
            def _():
                pltpu.sync_copy(z_h.at[rsl_main], acc.at[rsl_main])

            @pl.when(s == NS - 1)
            def _():
                pltpu.sync_copy(z_h.at[rsl_last], acc.at[rsl_last])

            plsc.subcore_barrier()

            def gbody(q, carry):
                r0 = s * (E_PER_TILE // K_EDGE) + q * QCH
                pltpu.sync_copy(cols_h.at[pl.ds(r0, QCH)], cols_v)
                pltpu.sync_copy(
                    rows_h.at[pl.ds(r0 * SUB, QCH * SUB)], rows_v
                )
                pltpu.sync_copy(vals_h.at[pl.ds(r0, QCH)], vals_v)

                def cbody(t, c1):
                    pltpu.async_copy(
                        x_h.at[cols_v.at[t]], stage, sem.at[0]
                    ).wait()

                    def sbody(gg, c2):
                        val16 = vals_v[t, pl.ds(gg * LANES, LANES)]
                        for i in range(LANES):
                            v = lax.gather(
                                val16,
                                jnp.full((LANES, 1), i, jnp.int32),
                                dimension_numbers=lax.GatherDimensionNumbers(
                                    offset_dims=(),
                                    collapsed_slice_dims=(0,),
                                    start_index_map=(0,),
                                ),
                                slice_sizes=(1,),
                                mode=lax.GatherScatterMode.PROMISE_IN_BOUNDS,
                            )
                            e = gg * LANES + i
                            for h in range(CCH // LANES):
                                csl = pl.ds(h * LANES, LANES)
                                stage[e, csl] = stage[e, csl] * v
                        return c2

                    lax.fori_loop(0, K_EDGE // LANES, sbody, 0)

                    hs = [
                        pltpu.async_copy(
                            stage.at[pl.ds(j * 128, 128)],
                            acc.at[rows_v.at[t * SUB + j]],
                            sem.at[1], add=True,
                        )
                        for j in range(SUB)
                    ]
                    for h_ in hs:
                        h_.wait()
                    return c1

                lax.fori_loop(0, QCH, cbody, 0)
                return carry

            lax.fori_loop(0, G_BIG // QCH, gbody, 0)
            plsc.subcore_barrier()

            @pl.when(s < NS - 1)
            def _():
                pltpu.sync_copy(acc.at[rsl_main], out_h.at[rsl_main])

            @pl.when(s == NS - 1)
            def _():
                pltpu.sync_copy(acc.at[rsl_last], out_h.at[rsl_last])

        c = lax.axis_index("c")

        @pl.when(c == 0)
        def _():
            run(xa_h, outa_h)

        @pl.when(c == 1)
        def _():
            run(xb_h, outb_h)

    return spmm_kernel(xa, xb, cols2d, rows2d, vals1d, zeros_nc)


def kernel(adj_indices, adj_values, user_embeddings, item_embeddings, v_feat, t_feat, id_prompt, v_prompt, t_prompt, W_id, b_id, W_v, b_v, W_t, b_t):
    pad = E_PAD - NNZ
    rows = jnp.concatenate([adj_indices[0], jnp.zeros((pad,), jnp.int32)])
    cols = jnp.concatenate([adj_indices[1], jnp.zeros((pad,), jnp.int32)])
    vals = jnp.concatenate([adj_values, jnp.zeros((pad,), jnp.float32)])
    rows2d = rows.reshape(E_PAD // 128, 128)
    cols2d = cols.reshape(E_PAD // K_EDGE, K_EDGE)
    vals2d = vals.reshape(E_PAD // K_EDGE, K_EDGE)
    zeros_nc = jnp.zeros((N_NODES, CCH), jnp.float32)

    u_chunks = _user_part(user_embeddings, id_prompt, v_prompt, t_prompt)
    i_id = _item_branch(item_embeddings, W_id, b_id)
    i_v = _item_branch(v_feat, W_v, b_v)
    i_t = _item_branch(t_feat, W_t, b_t)
    i_chunks = [i_id[0], i_id[1], i_v[0], i_v[1], i_t[0], i_t[1]]
    ego_chunks = [
        jnp.concatenate([u, it], axis=0) for u, it in zip(u_chunks, i_chunks)
    ]
    ego = jnp.concatenate(ego_chunks, axis=1)

    acc = ego
    y_chunks = ego_chunks
    for _ in range(N_LAYERS):
        s_chunks = [None] * NCH
        for k in range(NCH // NC):
            sa, sb = _spmm_pair(
                y_chunks[k], y_chunks[k + NCH // NC],
                cols2d, rows2d, vals2d, zeros_nc,
            )
            s_chunks[k] = sa
            s_chunks[k + NCH // NC] = sb
        y_chunks, acc = _cosine_acc(s_chunks, ego_chunks, acc)
    return acc[:N_USER], acc[N_USER:]


# speedup vs baseline: 1.1998x; 1.0488x over previous
import functools

import jax
import jax.numpy as jnp
from jax import lax
from jax.experimental import pallas as pl
from jax.experimental.pallas import tpu as pltpu
from jax.experimental.pallas import tpu_sc as plsc

N_USER = 30000
N_ITEM = 20000
N_NODES = N_USER + N_ITEM
NNZ = 800000
DIM = 64
NBRANCH = 3
DIMC = DIM * NBRANCH
CCH = 32
NCH = DIMC // CCH
N_LAYERS = 4
EPS = 1e-8

NC = 2
NS = 16
LANES = 16

UNIT = 256
QU = 8
W_PER_Q = QU // 2
E_PER_TILE = 51200
NQ = E_PER_TILE // (UNIT * QU)
E_PAD = E_PER_TILE * NS
ROWS_MAIN = 3128
ROWS_LAST = N_NODES - (NS - 1) * ROWS_MAIN


def _item_branch(feat, W, b):
    M, Kd = feat.shape
    BM = 400

    def body(f_ref, w_ref, b_ref, o1_ref, o2_ref):
        t = jnp.tanh(
            jnp.dot(f_ref[...], w_ref[...], preferred_element_type=jnp.float32)
            + b_ref[...]
        )
        o1_ref[...] = t[:, :CCH]
        o2_ref[...] = t[:, CCH:]

    ospec = pl.BlockSpec((BM, CCH), lambda i: (i, 0))
    return pl.pallas_call(
        body,
        grid=(M // BM,),
        in_specs=[
            pl.BlockSpec((BM, Kd), lambda i: (i, 0)),
            pl.BlockSpec((Kd, DIM), lambda i: (0, 0)),
            pl.BlockSpec((1, DIM), lambda i: (0, 0)),
        ],
        out_specs=[ospec, ospec],
        out_shape=[jax.ShapeDtypeStruct((M, CCH), jnp.float32)] * 2,
    )(feat, W, b.reshape(1, DIM))


def _user_part(user_embeddings, id_prompt, v_prompt, t_prompt):
    BM = 400

    def body(u_ref, pid_ref, pv_ref, pt_ref, *o_refs):
        u = u_ref[...]
        for b, p_ref in enumerate((pid_ref, pv_ref, pt_ref)):
            ps = jnp.sum(p_ref[...], axis=0, keepdims=True)
            o_refs[2 * b][...] = u[:, :CCH] + ps[:, :CCH]
            o_refs[2 * b + 1][...] = u[:, CCH:] + ps[:, CCH:]

    pspec = pl.BlockSpec((3, DIM), lambda i: (0, 0))
    ospec = pl.BlockSpec((BM, CCH), lambda i: (i, 0))
    return pl.pallas_call(
        body,
        grid=(N_USER // BM,),
        in_specs=[pl.BlockSpec((BM, DIM), lambda i: (i, 0)), pspec, pspec, pspec],
        out_specs=[ospec] * NCH,
        out_shape=[jax.ShapeDtypeStruct((N_USER, CCH), jnp.float32)] * NCH,
    )(user_embeddings, id_prompt, v_prompt, t_prompt)


def _cosine_acc(y_chunks, ego_chunks, acc):
    BM = 400

    def body(*refs):
        y_refs = refs[0:NCH]
        e_refs = refs[NCH:2 * NCH]
        a_ref = refs[2 * NCH]
        yw_refs = refs[2 * NCH + 1:2 * NCH + 1 + NCH]
        ao_ref = refs[2 * NCH + 1 + NCH]
        parts = []
        for b in range(NBRANCH):
            ys = jnp.concatenate([y_refs[2 * b][...], y_refs[2 * b + 1][...]], axis=1)
            es = jnp.concatenate([e_refs[2 * b][...], e_refs[2 * b + 1][...]], axis=1)
            dot = jnp.sum(ys * es, axis=1, keepdims=True)
            ny = jnp.sqrt(jnp.sum(ys * ys, axis=1, keepdims=True))
            ne = jnp.sqrt(jnp.sum(es * es, axis=1, keepdims=True))
            w = dot / (jnp.maximum(ny, EPS) * jnp.maximum(ne, EPS))
            yw = w * ys
            yw_refs[2 * b][...] = yw[:, :CCH]
            yw_refs[2 * b + 1][...] = yw[:, CCH:]
            parts.append(yw)
        ao_ref[...] = a_ref[...] + jnp.concatenate(parts, axis=1)

    cspec = pl.BlockSpec((BM, CCH), lambda i: (i, 0))
    aspec = pl.BlockSpec((BM, DIMC), lambda i: (i, 0))
    outs = pl.pallas_call(
        body,
        grid=(N_NODES // BM,),
        in_specs=[cspec] * NCH + [cspec] * NCH + [aspec],
        out_specs=[cspec] * NCH + [aspec],
        out_shape=[jax.ShapeDtypeStruct((N_NODES, CCH), jnp.float32)] * NCH
        + [jax.ShapeDtypeStruct((N_NODES, DIMC), jnp.float32)],
    )(*y_chunks, *ego_chunks, acc)
    return list(outs[:NCH]), outs[NCH]


def _spmm_pair(xa, xb, cols2d, rows2d, vals1d, zeros_nc):
    mesh = plsc.VectorSubcoreMesh(
        core_axis_name="c", subcore_axis_name="s", num_cores=NC, num_subcores=NS
    )

    @functools.partial(
        pl.kernel,
        out_type=[jax.ShapeDtypeStruct((N_NODES, CCH), jnp.float32)] * 2,
        mesh=mesh,
        compiler_params=pltpu.CompilerParams(use_tc_tiling_on_sc=False),
        scratch_types=[
            pltpu.VMEM_SHARED((N_NODES, CCH), jnp.float32),
            pltpu.VMEM((QU, UNIT), jnp.int32),
            pltpu.VMEM((QU * 2, 128), jnp.int32),
            pltpu.VMEM((QU, UNIT), jnp.float32),
            pltpu.VMEM((2 * UNIT, CCH), jnp.float32),
            pltpu.SemaphoreType.DMA((2,)),
            pltpu.SemaphoreType.DMA((2,)),
        ],
    )
    def spmm_kernel(xa_h, xb_h, cols_h, rows_h, vals_h, z_h, outa_h, outb_h,
                    acc, cols_v, rows_v, vals_v, stage, sem_g, sem_s):
        s = lax.axis_index("s")

        def run(x_h, out_h):
            rsl_main = pl.ds(s * ROWS_MAIN, ROWS_MAIN)
            rsl_last = pl.ds((NS - 1) * ROWS_MAIN, ROWS_LAST)

            @pl.when(s < NS - 1)
            def _():
                pltpu.sync_copy(z_h.at[rsl_main], acc.at[rsl_main])

            @pl.when(s == NS - 1)
            def _():
                pltpu.sync_copy(z_h.at[rsl_last], acc.at[rsl_last])

            plsc.subcore_barrier()

            def scale(b, vrow):
                def sbody(gg, c2):
                    val16 = vals_v[vrow, pl.ds(gg * LANES, LANES)]
                    for i in range(LANES):
                        v = lax.gather(
                            val16,
                            jnp.full((LANES, 1), i, jnp.int32),
                            dimension_numbers=lax.GatherDimensionNumbers(
                                offset_dims=(),
                                collapsed_slice_dims=(0,),
                                start_index_map=(0,),
                            ),
                            slice_sizes=(1,),
                            mode=lax.GatherScatterMode.PROMISE_IN_BOUNDS,
                        )
                        e = b * UNIT + gg * LANES + i
                        for h in range(CCH // LANES):
                            csl = pl.ds(h * LANES, LANES)
                            stage[e, csl] = stage[e, csl] * v
                    return c2

                lax.fori_loop(0, UNIT // LANES, sbody, 0)

            def fire_g(b, urow):
                return pltpu.async_copy(
                    x_h.at[cols_v.at[urow]],
                    stage.at[pl.ds(b * UNIT, UNIT)],
                    sem_g.at[b],
                )

            def wait_g(b):
                pltpu.make_async_copy(
                    x_h.at[cols_v.at[0]],
                    stage.at[pl.ds(b * UNIT, UNIT)],
                    sem_g.at[b],
                ).wait()

            def fire_s(b, urow):
                return [
                    pltpu.async_copy(
                        stage.at[pl.ds(b * UNIT + j * 128, 128)],
                        acc.at[rows_v.at[2 * urow + j]],
                        sem_s.at[b], add=True,
                    )
                    for j in range(2)
                ]

            def wait_s(b):
                for j in range(2):
                    pltpu.make_async_copy(
                        stage.at[pl.ds(b * UNIT + j * 128, 128)],
                        acc.at[rows_v.at[j]],
                        sem_s.at[b],
                    ).wait()

            def qbody(q, carry):
                @pl.when(q > 0)
                def _():
                    wait_s(1)

                base = s * (E_PER_TILE // UNIT) + q * QU
                pltpu.sync_copy(cols_h.at[pl.ds(base, QU)], cols_v)
                pltpu.sync_copy(rows_h.at[pl.ds(2 * base, 2 * QU)], rows_v)
                pltpu.sync_copy(vals_h.at[pl.ds(base, QU)], vals_v)
                fire_g(0, 0)

                def wbody(w, c1):
                    u0 = 2 * w
                    u1 = 2 * w + 1
                    wait_g(0)

                    @pl.when(w > 0)
                    def _():
                        wait_s(1)

                    fire_g(1, u1)
                    scale(0, u0)
                    fire_s(0, u0)
                    wait_g(1)
                    scale(1, u1)
                    fire_s(1, u1)
                    wait_s(0)

                    @pl.when(w < W_PER_Q - 1)
                    def _():
                        fire_g(0, u0 + 2)

                    return c1

                lax.fori_loop(0, W_PER_Q, wbody, 0)
                return carry

            lax.fori_loop(0, NQ, qbody, 0)
            wait_s(1)
            plsc.subcore_barrier()

            @pl.when(s < NS - 1)
            def _():
                pltpu.sync_copy(acc.at[rsl_main], out_h.at[rsl_main])

            @pl.when(s == NS - 1)
            def _():
                pltpu.sync_copy(acc.at[rsl_last], out_h.at[rsl_last])

        c = lax.axis_index("c")

        @pl.when(c == 0)
        def _():
            run(xa_h, outa_h)

        @pl.when(c == 1)
        def _():
            run(xb_h, outb_h)

    return spmm_kernel(xa, xb, cols2d, rows2d, vals1d, zeros_nc)


def kernel(adj_indices, adj_values, user_embeddings, item_embeddings, v_feat, t_feat, id_prompt, v_prompt, t_prompt, W_id, b_id, W_v, b_v, W_t, b_t):
    pad = E_PAD - NNZ
    rows = jnp.concatenate([adj_indices[0], jnp.zeros((pad,), jnp.int32)])
    cols = jnp.concatenate([adj_indices[1], jnp.zeros((pad,), jnp.int32)])
    vals = jnp.concatenate([adj_values, jnp.zeros((pad,), jnp.float32)])
    rows2d = rows.reshape(E_PAD // 128, 128)
    cols2d = cols.reshape(E_PAD // UNIT, UNIT)
    vals2d = vals.reshape(E_PAD // UNIT, UNIT)
    zeros_nc = jnp.zeros((N_NODES, CCH), jnp.float32)

    u_chunks = _user_part(user_embeddings, id_prompt, v_prompt, t_prompt)
    i_id = _item_branch(item_embeddings, W_id, b_id)
    i_v = _item_branch(v_feat, W_v, b_v)
    i_t = _item_branch(t_feat, W_t, b_t)
    i_chunks = [i_id[0], i_id[1], i_v[0], i_v[1], i_t[0], i_t[1]]
    ego_chunks = [
        jnp.concatenate([u, it], axis=0) for u, it in zip(u_chunks, i_chunks)
    ]
    ego = jnp.concatenate(ego_chunks, axis=1)

    acc = ego
    y_chunks = ego_chunks
    for _ in range(N_LAYERS):
        s_chunks = [None] * NCH
        for k in range(NCH // NC):
            sa, sb = _spmm_pair(
                y_chunks[k], y_chunks[k + NCH // NC],
                cols2d, rows2d, vals2d, zeros_nc,
            )
            s_chunks[k] = sa
            s_chunks[k + NCH // NC] = sb
        y_chunks, acc = _cosine_acc(s_chunks, ego_chunks, acc)
    return acc[:N_USER], acc[N_USER:]


# speedup vs baseline: 1.2292x; 1.0245x over previous
import functools

import jax
import jax.numpy as jnp
from jax import lax
from jax.experimental import pallas as pl
from jax.experimental.pallas import tpu as pltpu
from jax.experimental.pallas import tpu_sc as plsc

N_USER = 30000
N_ITEM = 20000
N_NODES = N_USER + N_ITEM
NNZ = 800000
DIM = 64
NBRANCH = 3
DIMC = DIM * NBRANCH
CCH = 32
NCH = DIMC // CCH
N_LAYERS = 4
EPS = 1e-8

NC = 2
NS = 16
LANES = 16

UNIT = 256
QU = 10
W_PER_Q = QU // 2
E_PER_TILE = 51200
NQ = E_PER_TILE // (UNIT * QU)
E_PAD = E_PER_TILE * NS
ROWS_MAIN = 3128
ROWS_LAST = N_NODES - (NS - 1) * ROWS_MAIN


def _item_branch(feat, W, b):
    M, Kd = feat.shape
    BM = 400

    def body(f_ref, w_ref, b_ref, o1_ref, o2_ref):
        t = jnp.tanh(
            jnp.dot(f_ref[...], w_ref[...], preferred_element_type=jnp.float32)
            + b_ref[...]
        )
        o1_ref[...] = t[:, :CCH]
        o2_ref[...] = t[:, CCH:]

    ospec = pl.BlockSpec((BM, CCH), lambda i: (i, 0))
    return pl.pallas_call(
        body,
        grid=(M // BM,),
        in_specs=[
            pl.BlockSpec((BM, Kd), lambda i: (i, 0)),
            pl.BlockSpec((Kd, DIM), lambda i: (0, 0)),
            pl.BlockSpec((1, DIM), lambda i: (0, 0)),
        ],
        out_specs=[ospec, ospec],
        out_shape=[jax.ShapeDtypeStruct((M, CCH), jnp.float32)] * 2,
    )(feat, W, b.reshape(1, DIM))


def _user_part(user_embeddings, id_prompt, v_prompt, t_prompt):
    BM = 400

    def body(u_ref, pid_ref, pv_ref, pt_ref, *o_refs):
        u = u_ref[...]
        for b, p_ref in enumerate((pid_ref, pv_ref, pt_ref)):
            ps = jnp.sum(p_ref[...], axis=0, keepdims=True)
            o_refs[2 * b][...] = u[:, :CCH] + ps[:, :CCH]
            o_refs[2 * b + 1][...] = u[:, CCH:] + ps[:, CCH:]

    pspec = pl.BlockSpec((3, DIM), lambda i: (0, 0))
    ospec = pl.BlockSpec((BM, CCH), lambda i: (i, 0))
    return pl.pallas_call(
        body,
        grid=(N_USER // BM,),
        in_specs=[pl.BlockSpec((BM, DIM), lambda i: (i, 0)), pspec, pspec, pspec],
        out_specs=[ospec] * NCH,
        out_shape=[jax.ShapeDtypeStruct((N_USER, CCH), jnp.float32)] * NCH,
    )(user_embeddings, id_prompt, v_prompt, t_prompt)


def _cosine_acc(y_chunks, ego_chunks, acc):
    BM = 400

    def body(*refs):
        y_refs = refs[0:NCH]
        e_refs = refs[NCH:2 * NCH]
        a_ref = refs[2 * NCH]
        yw_refs = refs[2 * NCH + 1:2 * NCH + 1 + NCH]
        ao_ref = refs[2 * NCH + 1 + NCH]
        parts = []
        for b in range(NBRANCH):
            ys = jnp.concatenate([y_refs[2 * b][...], y_refs[2 * b + 1][...]], axis=1)
            es = jnp.concatenate([e_refs[2 * b][...], e_refs[2 * b + 1][...]], axis=1)
            dot = jnp.sum(ys * es, axis=1, keepdims=True)
            ny = jnp.sqrt(jnp.sum(ys * ys, axis=1, keepdims=True))
            ne = jnp.sqrt(jnp.sum(es * es, axis=1, keepdims=True))
            w = dot / (jnp.maximum(ny, EPS) * jnp.maximum(ne, EPS))
            yw = w * ys
            yw_refs[2 * b][...] = yw[:, :CCH]
            yw_refs[2 * b + 1][...] = yw[:, CCH:]
            parts.append(yw)
        ao_ref[...] = a_ref[...] + jnp.concatenate(parts, axis=1)

    cspec = pl.BlockSpec((BM, CCH), lambda i: (i, 0))
    aspec = pl.BlockSpec((BM, DIMC), lambda i: (i, 0))
    outs = pl.pallas_call(
        body,
        grid=(N_NODES // BM,),
        in_specs=[cspec] * NCH + [cspec] * NCH + [aspec],
        out_specs=[cspec] * NCH + [aspec],
        out_shape=[jax.ShapeDtypeStruct((N_NODES, CCH), jnp.float32)] * NCH
        + [jax.ShapeDtypeStruct((N_NODES, DIMC), jnp.float32)],
    )(*y_chunks, *ego_chunks, acc)
    return list(outs[:NCH]), outs[NCH]


def _spmm_pair(xa, xb, cols2d, rows2d, vals1d, zeros_nc):
    mesh = plsc.VectorSubcoreMesh(
        core_axis_name="c", subcore_axis_name="s", num_cores=NC, num_subcores=NS
    )

    @functools.partial(
        pl.kernel,
        out_type=[jax.ShapeDtypeStruct((N_NODES, CCH), jnp.float32)] * 2,
        mesh=mesh,
        compiler_params=pltpu.CompilerParams(use_tc_tiling_on_sc=False),
        scratch_types=[
            pltpu.VMEM_SHARED((N_NODES, CCH), jnp.float32),
            pltpu.VMEM((QU, UNIT), jnp.int32),
            pltpu.VMEM((QU, UNIT), jnp.int32),
            pltpu.VMEM((QU, UNIT), jnp.float32),
            pltpu.VMEM((2 * UNIT, CCH), jnp.float32),
            pltpu.SemaphoreType.DMA((2,)),
            pltpu.SemaphoreType.DMA((2,)),
        ],
    )
    def spmm_kernel(xa_h, xb_h, cols_h, rows_h, vals_h, z_h, outa_h, outb_h,
                    acc, cols_v, rows_v, vals_v, stage, sem_g, sem_s):
        s = lax.axis_index("s")

        def run(x_h, out_h):
            rsl_main = pl.ds(s * ROWS_MAIN, ROWS_MAIN)
            rsl_last = pl.ds((NS - 1) * ROWS_MAIN, ROWS_LAST)

            @pl.when(s < NS - 1)
            def _():
                pltpu.sync_copy(z_h.at[rsl_main], acc.at[rsl_main])

            @pl.when(s == NS - 1)
            def _():
                pltpu.sync_copy(z_h.at[rsl_last], acc.at[rsl_last])

            plsc.subcore_barrier()

            def scale(b, vrow):
                def sbody(gg, c2):
                    val16 = vals_v[vrow, pl.ds(gg * LANES, LANES)]
                    for i in range(LANES):
                        v = lax.gather(
                            val16,
                            jnp.full((LANES, 1), i, jnp.int32),
                            dimension_numbers=lax.GatherDimensionNumbers(
                                offset_dims=(),
                                collapsed_slice_dims=(0,),
                                start_index_map=(0,),
                            ),
                            slice_sizes=(1,),
                            mode=lax.GatherScatterMode.PROMISE_IN_BOUNDS,
                        )
                        e = b * UNIT + gg * LANES + i
                        for h in range(CCH // LANES):
                            csl = pl.ds(h * LANES, LANES)
                            stage[e, csl] = stage[e, csl] * v
                    return c2

                lax.fori_loop(0, UNIT // LANES, sbody, 0)

            def fire_g(b, urow):
                return pltpu.async_copy(
                    x_h.at[cols_v.at[urow]],
                    stage.at[pl.ds(b * UNIT, UNIT)],
                    sem_g.at[b],
                )

            def wait_g(b):
                pltpu.make_async_copy(
                    x_h.at[cols_v.at[0]],
                    stage.at[pl.ds(b * UNIT, UNIT)],
                    sem_g.at[b],
                ).wait()

            def fire_s(b, urow):
                return pltpu.async_copy(
                    stage.at[pl.ds(b * UNIT, UNIT)],
                    acc.at[rows_v.at[urow]],
                    sem_s.at[b], add=True,
                )

            def wait_s(b):
                pltpu.make_async_copy(
                    stage.at[pl.ds(b * UNIT, UNIT)],
                    acc.at[rows_v.at[0]],
                    sem_s.at[b],
                ).wait()

            def qbody(q, carry):
                @pl.when(q > 0)
                def _():
                    wait_s(1)

                base = s * (E_PER_TILE // UNIT) + q * QU
                pltpu.sync_copy(cols_h.at[pl.ds(base, QU)], cols_v)
                pltpu.sync_copy(rows_h.at[pl.ds(base, QU)], rows_v)
                pltpu.sync_copy(vals_h.at[pl.ds(base, QU)], vals_v)
                fire_g(0, 0)

                def wbody(w, c1):
                    u0 = 2 * w
                    u1 = 2 * w + 1
                    wait_g(0)

                    @pl.when(w > 0)
                    def _():
                        wait_s(1)

                    fire_g(1, u1)
                    scale(0, u0)
                    fire_s(0, u0)
                    wait_g(1)
                    scale(1, u1)
                    fire_s(1, u1)
                    wait_s(0)

                    @pl.when(w < W_PER_Q - 1)
                    def _():
                        fire_g(0, u0 + 2)

                    return c1

                lax.fori_loop(0, W_PER_Q, wbody, 0)
                return carry

            lax.fori_loop(0, NQ, qbody, 0)
            wait_s(1)
            plsc.subcore_barrier()

            @pl.when(s < NS - 1)
            def _():
                pltpu.sync_copy(acc.at[rsl_main], out_h.at[rsl_main])

            @pl.when(s == NS - 1)
            def _():
                pltpu.sync_copy(acc.at[rsl_last], out_h.at[rsl_last])

        c = lax.axis_index("c")

        @pl.when(c == 0)
        def _():
            run(xa_h, outa_h)

        @pl.when(c == 1)
        def _():
            run(xb_h, outb_h)

    return spmm_kernel(xa, xb, cols2d, rows2d, vals1d, zeros_nc)


def kernel(adj_indices, adj_values, user_embeddings, item_embeddings, v_feat, t_feat, id_prompt, v_prompt, t_prompt, W_id, b_id, W_v, b_v, W_t, b_t):
    pad = E_PAD - NNZ
    rows = jnp.concatenate([adj_indices[0], jnp.zeros((pad,), jnp.int32)])
    cols = jnp.concatenate([adj_indices[1], jnp.zeros((pad,), jnp.int32)])
    vals = jnp.concatenate([adj_values, jnp.zeros((pad,), jnp.float32)])
    rows2d = rows.reshape(E_PAD // UNIT, UNIT)
    cols2d = cols.reshape(E_PAD // UNIT, UNIT)
    vals2d = vals.reshape(E_PAD // UNIT, UNIT)
    zeros_nc = jnp.zeros((N_NODES, CCH), jnp.float32)

    u_chunks = _user_part(user_embeddings, id_prompt, v_prompt, t_prompt)
    i_id = _item_branch(item_embeddings, W_id, b_id)
    i_v = _item_branch(v_feat, W_v, b_v)
    i_t = _item_branch(t_feat, W_t, b_t)
    i_chunks = [i_id[0], i_id[1], i_v[0], i_v[1], i_t[0], i_t[1]]
    ego_chunks = [
        jnp.concatenate([u, it], axis=0) for u, it in zip(u_chunks, i_chunks)
    ]
    ego = jnp.concatenate(ego_chunks, axis=1)

    acc = ego
    y_chunks = ego_chunks
    for _ in range(N_LAYERS):
        s_chunks = [None] * NCH
        for k in range(NCH // NC):
            sa, sb = _spmm_pair(
                y_chunks[k], y_chunks[k + NCH // NC],
                cols2d, rows2d, vals2d, zeros_nc,
            )
            s_chunks[k] = sa
            s_chunks[k + NCH // NC] = sb
        y_chunks, acc = _cosine_acc(s_chunks, ego_chunks, acc)
    return acc[:N_USER], acc[N_USER:]
